# baseline (speedup 1.0000x reference)
import functools
import jax
import jax.numpy as jnp
from jax import lax
from jax.experimental import pallas as pl
from jax.experimental.pallas import tpu as pltpu, tpu_sc as plsc

GRID = 0.0078125
SCALE_MULT = 0.05
N = 8192
P = 1024
C = 18

NC = 2
NS = 16
NW = NC * NS
NCELL = 512
CPW = NCELL // NW
CAP = 64
LSTR = 80
PCAP = 384
HITCAP = 320

_mesh = plsc.VectorSubcoreMesh(core_axis_name="c", subcore_axis_name="s")


def _sload(ref, i):
    return ref[pl.ds(i, 16)][0]


def _sstore(ref, i, val, dtype):
    lane0 = lax.iota(jnp.int32, 16) == 0
    plsc.store_scatter(ref, [jnp.full((16,), i, jnp.int32)],
                       jnp.full((16,), val, dtype), mask=lane0)


def _popcnt(m):
    return plsc.all_reduce_population_count(m)[0]


@functools.partial(
    pl.kernel,
    out_type=jax.ShapeDtypeStruct((N * 32,), jnp.float32),
    mesh=_mesh,
    compiler_params=pltpu.CompilerParams(needs_layout_passes=False),
    scratch_types=[
        pltpu.VMEM((3 * P + 16,), jnp.float32),
        pltpu.VMEM((9 * P + 16,), jnp.float32),
        pltpu.VMEM((P + 16,), jnp.float32),
        pltpu.VMEM((3 * P + 16,), jnp.int32),
        pltpu.VMEM((P + 16,), jnp.int32),
        pltpu.VMEM((P * 16,), jnp.float32),
        pltpu.VMEM((P * 8 + 16,), jnp.float32),
        pltpu.VMEM((2 * P,), jnp.int32),
        pltpu.VMEM((8 * P,), jnp.int32),
        pltpu.VMEM((P + 16,), jnp.int32),
        pltpu.VMEM((3 * N + 16,), jnp.int32),
        pltpu.VMEM((2048,), jnp.int32),
        pltpu.VMEM((3 * N + 16,), jnp.float32),
        pltpu.VMEM((CPW * LSTR,), jnp.int32),
        pltpu.VMEM((CPW + 16,), jnp.int32),
        pltpu.VMEM((PCAP + 32,), jnp.int32),
        pltpu.VMEM((PCAP + 32,), jnp.int32),
        pltpu.VMEM((PCAP * 32 + 32,), jnp.float32),
        pltpu.VMEM((HITCAP + 32,), jnp.float32),
        pltpu.VMEM((HITCAP + 32,), jnp.int32),
        pltpu.VMEM((HITCAP + 32,), jnp.int32),
        pltpu.SemaphoreType.DMA,
    ],
)
def _sc_agg(mu3_hbm, cov9_hbm, op_hbm, mint3_hbm, rad_hbm, semA_hbm,
            semB_hbm, p3_hbm, pint3_hbm, pc_hbm, out_hbm,
            mu3_v, cov9_v, op_v, mint3_v, rad_v, semA_v, semB_v,
            ym_v, zm_v, xl_v, pint3_v, cidc_v, p3_v, ll_v, lc_v,
            pidf_v, pcidf_v, stag_v, hw_v, hg_v, hil_v, dsem):
    cidx = lax.axis_index("c")
    sidx = lax.axis_index("s")
    wid = sidx * NC + cidx
    mycx = wid // 4
    cy0 = (wid * 2) % 8

    pltpu.sync_copy(mu3_hbm, mu3_v.at[pl.ds(0, 3 * P)])
    pltpu.sync_copy(cov9_hbm, cov9_v.at[pl.ds(0, 9 * P)])
    pltpu.sync_copy(op_hbm, op_v.at[pl.ds(0, P)])
    pltpu.sync_copy(mint3_hbm, mint3_v.at[pl.ds(0, 3 * P)])
    pltpu.sync_copy(rad_hbm, rad_v.at[pl.ds(0, P)])
    pltpu.sync_copy(semA_hbm, semA_v)
    pltpu.sync_copy(semB_hbm, semB_v.at[pl.ds(0, P * 8)])
    pltpu.sync_copy(pint3_hbm, pint3_v.at[pl.ds(0, 3 * N)])
    pltpu.sync_copy(p3_hbm, p3_v.at[pl.ds(0, 3 * N)])

    iota = lax.iota(jnp.int32, 16)

    def _xscan(j, off):
        g3 = (iota + j * 16) * 3
        mix = plsc.load_gather(mint3_v, [g3])
        rr = rad_v[pl.ds(j * 16, 16)]
        m = ((lax.shift_right_arithmetic(mix - rr, 4) <= mycx)
             & (mycx <= lax.shift_right_arithmetic(mix + rr, 4)))
        mi = m.astype(jnp.int32)
        pos = plsc.cumsum(mi) - mi + off
        plsc.store_scatter(xl_v, [pos], iota + j * 16, mask=m)
        return jnp.minimum(off + _popcnt(m), P)
    xcnt = lax.fori_loop(0, P // 16, _xscan, jnp.int32(0))

    def _yz(j, _):
        g3 = (iota + j * 16) * 3
        miy = plsc.load_gather(mint3_v, [g3 + 1])
        miz = plsc.load_gather(mint3_v, [g3 + 2])
        rr = rad_v[pl.ds(j * 16, 16)]
        ylo = lax.shift_right_arithmetic(miy - rr, 4)
        yhi = lax.shift_right_arithmetic(miy + rr, 4)
        ym_v[pl.ds(j * 16, 16)] = ((ylo <= cy0) & (cy0 <= yhi)).astype(jnp.int32)
        ym_v[pl.ds(P + j * 16, 16)] = (
            (ylo <= cy0 + 1) & (cy0 + 1 <= yhi)).astype(jnp.int32)
        zlo = lax.shift_right_arithmetic(miz - rr, 4)
        zhi = lax.shift_right_arithmetic(miz + rr, 4)
        for q in range(8):
            zm_v[pl.ds(q * P + j * 16, 16)] = (
                (zlo <= q) & (q <= zhi)).astype(jnp.int32)
        return _
    lax.fori_loop(0, P // 16, _yz, 0)

    nxv = (xcnt + 15) // 16

    def _cell_body(k, _):
        yrow = (k // 8) * P
        zrow = (k - (k // 8) * 8) * P

        def _j(j, off):
            gid = xl_v[pl.ds(j * 16, 16)]
            lane_ok = (iota + j * 16) < xcnt
            gid = jnp.where(lane_ok, gid, 0)
            ym = plsc.load_gather(ym_v, [gid + yrow])
            zm = plsc.load_gather(zm_v, [gid + zrow])
            m = lane_ok & ((ym & zm) == 1)
            mi = m.astype(jnp.int32)
            pos = plsc.cumsum(mi) - mi + (k * LSTR + off)
            plsc.store_scatter(ll_v, [pos], gid, mask=m)
            return jnp.minimum(off + _popcnt(m), CAP)
        cnt = lax.fori_loop(0, nxv, _j, jnp.int32(0))
        _sstore(lc_v, k, cnt, jnp.int32)
        return _
    lax.fori_loop(0, CPW, _cell_body, 0)

    npts = jnp.int32(0)
    for b in range(N // 2048):
        pltpu.sync_copy(pc_hbm.at[pl.ds(b * 2048, 2048)], cidc_v)

        def _pt_scan(v, np_, _b=b):
            cv = cidc_v[pl.ds(v * 16, 16)]
            m = (cv >> 4) == wid
            ids = iota + (_b * 2048 + v * 16)
            mi = m.astype(jnp.int32)
            pos = plsc.cumsum(mi) - mi + np_
            plsc.store_scatter(pidf_v, [pos], ids, mask=m)
            plsc.store_scatter(pcidf_v, [pos], cv, mask=m)
            return jnp.minimum(np_ + _popcnt(m), PCAP)
        npts = lax.fori_loop(0, 2048 // 16, _pt_scan, npts)

    def _pt_body(i, hoff):
        pid = _sload(pidf_v, i)
        pcid = _sload(pcidf_v, i)
        k = pcid - wid * CPW
        cnt = _sload(lc_v, k)
        p3 = 3 * pid
        pix = _sload(pint3_v, p3)
        piy = _sload(pint3_v, p3 + 1)
        piz = _sload(pint3_v, p3 + 2)
        stag_v[pl.ds(i * 32, 16)] = jnp.zeros((16,), jnp.float32)
        stag_v[pl.ds(i * 32 + 16, 16)] = jnp.zeros((16,), jnp.float32)
        nj = (cnt + 15) // 16

        def _j(j, hoff):
            idx = ll_v[pl.ds(k * LSTR + j * 16, 16)]
            valid = (iota + j * 16) < cnt
            gidx = jnp.where(valid, idx, 0)
            g3 = gidx * 3
            mix = plsc.load_gather(mint3_v, [g3])
            miy = plsc.load_gather(mint3_v, [g3 + 1])
            miz = plsc.load_gather(mint3_v, [g3 + 2])
            rr = plsc.load_gather(rad_v, [gidx])
            mask = (valid
                    & (jnp.abs(pix - mix) <= rr)
                    & (jnp.abs(piy - miy) <= rr)
                    & (jnp.abs(piz - miz) <= rr))
            c = _popcnt(mask)

            @pl.when(c > 0)
            def _store():
                mi = mask.astype(jnp.int32)
                pos = plsc.cumsum(mi) - mi + hoff
                plsc.store_scatter(hg_v, [pos], gidx, mask=mask)
                plsc.store_scatter(hil_v, [pos],
                                   jnp.full((16,), i, jnp.int32), mask=mask)
            return jnp.minimum(hoff + c, HITCAP)
        return lax.fori_loop(0, nj, _j, hoff)
    nhits = lax.fori_loop(0, npts, _pt_body, jnp.int32(0))

    nhv = (nhits + 15) // 16

    def _ev(hv, _):
        lane_ok = (iota + hv * 16) < nhits
        gidx = jnp.where(lane_ok, hg_v[pl.ds(hv * 16, 16)], 0)
        il = jnp.where(lane_ok, hil_v[pl.ds(hv * 16, 16)], 0)
        pidv = plsc.load_gather(pidf_v, [il])
        pp3 = pidv * 3
        px = plsc.load_gather(p3_v, [pp3])
        py = plsc.load_gather(p3_v, [pp3 + 1])
        pz = plsc.load_gather(p3_v, [pp3 + 2])
        g3 = gidx * 3
        g9 = gidx * 9
        mux = plsc.load_gather(mu3_v, [g3])
        muy = plsc.load_gather(mu3_v, [g3 + 1])
        muz = plsc.load_gather(mu3_v, [g3 + 2])
        c0 = plsc.load_gather(cov9_v, [g9])
        c1 = plsc.load_gather(cov9_v, [g9 + 4])
        c2 = plsc.load_gather(cov9_v, [g9 + 8])
        c3 = plsc.load_gather(cov9_v, [g9 + 1])
        c4 = plsc.load_gather(cov9_v, [g9 + 5])
        c5 = plsc.load_gather(cov9_v, [g9 + 2])
        opg = plsc.load_gather(op_v, [gidx])
        dx = px - mux
        dy = py - muy
        dz = pz - muz
        power = (-0.5 * (c0 * dx * dx + c1 * dy * dy + c2 * dz * dz)
                 - c3 * dx * dy - c4 * dy * dz - c5 * dx * dz)
        hw_v[pl.ds(hv * 16, 16)] = opg * jnp.exp(power)
        return _
    lax.fori_loop(0, nhv, _ev, 0)

    lo8 = iota < 8

    def _hit(h, _):
        wv = _sload(hw_v, h)
        g = _sload(hg_v, h)
        il = _sload(hil_v, h)
        a0 = stag_v[pl.ds(il * 32, 16)]
        stag_v[pl.ds(il * 32, 16)] = a0 + wv * semA_v[pl.ds(g * 16, 16)]
        a1 = stag_v[pl.ds(il * 32 + 16, 16)]
        sb = jnp.where(lo8, semB_v[pl.ds(g * 8, 16)], 0.0)
        stag_v[pl.ds(il * 32 + 16, 16)] = a1 + wv * sb
        return _
    lax.fori_loop(0, nhits, _hit, 0)

    def _out(i, _):
        pid = _sload(pidf_v, i)
        pltpu.async_copy(stag_v.at[pl.ds(i * 32, 32)],
                         out_hbm.at[pl.ds(pid * 32, 32)], dsem)
        return _
    lax.fori_loop(0, npts, _out, 0)

    def _drain(i, _):
        pltpu.make_async_copy(stag_v.at[pl.ds(0, 32)],
                              out_hbm.at[pl.ds(0, 32)], dsem).wait()
        return _
    lax.fori_loop(0, npts, _drain, 0)


def kernel(pts, means3D, opacities, semantics, scales, cov3D, metas, origin_use):
    p = pts[0]
    mu = means3D[0]
    op = opacities[0]
    sem = semantics[0]
    sc = scales[0]
    cov = cov3D[0]

    inv_g = 1.0 / GRID
    pint = jnp.floor((p - origin_use) * inv_g).astype(jnp.int32)
    mint = jnp.floor((mu - origin_use) * inv_g).astype(jnp.int32)
    radii = jnp.ceil(jnp.max(sc, axis=-1) * (SCALE_MULT * inv_g)).astype(jnp.int32)
    pcell = ((pint[:, 0] >> 4) * 64 + (pint[:, 1] >> 4) * 8 + (pint[:, 2] >> 4))

    semA = sem[:, :16].reshape(-1)
    semB = jnp.pad(sem[:, 16:], ((0, 0), (0, 6))).reshape(-1)

    out = _sc_agg(mu.reshape(-1), cov.reshape(-1), op, mint.reshape(-1),
                  radii, semA, semB, p.reshape(-1), pint.reshape(-1), pcell)
    return out.reshape(N, 32)[:, :C]


# speedup vs baseline: 1.1498x; 1.1498x over previous
import functools
import jax
import jax.numpy as jnp
from jax import lax
from jax.experimental import pallas as pl
from jax.experimental.pallas import tpu as pltpu, tpu_sc as plsc

GRID = 0.0078125
SCALE_MULT = 0.05
N = 8192
P = 1024
C = 18

NC = 2
NS = 16
NW = NC * NS
NCELL = 512
CPW = NCELL // NW
CAP = 64
LSTR = 80
PCAP = 384
HITCAP = 320

_mesh = plsc.VectorSubcoreMesh(core_axis_name="c", subcore_axis_name="s")


def _sload(ref, i):
    return ref[pl.ds(i, 16)][0]


def _sstore(ref, i, val, dtype):
    lane0 = lax.iota(jnp.int32, 16) == 0
    plsc.store_scatter(ref, [jnp.full((16,), i, jnp.int32)],
                       jnp.full((16,), val, dtype), mask=lane0)


@functools.partial(
    pl.kernel,
    out_type=jax.ShapeDtypeStruct((N * 32,), jnp.float32),
    mesh=_mesh,
    compiler_params=pltpu.CompilerParams(needs_layout_passes=False),
    scratch_types=[
        pltpu.VMEM((10 * P,), jnp.float32),
        pltpu.VMEM((4 * P + 16,), jnp.int32),
        pltpu.VMEM((P * 16,), jnp.float32),
        pltpu.VMEM((P * 8 + 16,), jnp.float32),
        pltpu.VMEM((2 * P,), jnp.int32),
        pltpu.VMEM((8 * P,), jnp.int32),
        pltpu.VMEM((P + 16,), jnp.int32),
        pltpu.VMEM((2048,), jnp.int32),
        pltpu.VMEM((3 * N + 16,), jnp.float32),
        pltpu.VMEM((CPW * LSTR,), jnp.int32),
        pltpu.VMEM((CPW + 16,), jnp.int32),
        pltpu.VMEM((PCAP + 32,), jnp.int32),
        pltpu.VMEM((PCAP + 32,), jnp.int32),
        pltpu.VMEM((PCAP * 32 + 32,), jnp.float32),
        pltpu.VMEM((HITCAP + 32,), jnp.float32),
        pltpu.VMEM((HITCAP + 32,), jnp.int32),
        pltpu.VMEM((HITCAP + 32,), jnp.int32),
        pltpu.SemaphoreType.DMA,
    ],
)
def _sc_agg(gf_hbm, gi_hbm, semA_hbm, semB_hbm, ptf_hbm, pc_hbm, out_hbm,
            gf_v, gi_v, semA_v, semB_v, ym_v, zm_v, xl_v, cidc_v, pw_v,
            ll_v, lc_v, pidf_v, pcidf_v, stag_v, hw_v, hg_v, hil_v, dsem):
    cidx = lax.axis_index("c")
    sidx = lax.axis_index("s")
    wid = sidx * NC + cidx
    mycx = wid // 4
    cy0 = (wid * 2) % 8

    pltpu.sync_copy(gf_hbm, gf_v)
    pltpu.sync_copy(gi_hbm, gi_v.at[pl.ds(0, 4 * P)])
    pltpu.sync_copy(semA_hbm, semA_v)
    pltpu.sync_copy(semB_hbm, semB_v.at[pl.ds(0, P * 8)])
    pltpu.sync_copy(ptf_hbm, pw_v.at[pl.ds(0, 3 * N)])

    def _xscan(j, off):
        mix = gi_v[pl.ds(j * 16, 16)]
        rr = gi_v[pl.ds(3 * P + j * 16, 16)]
        m = ((lax.shift_right_arithmetic(mix - rr, 4) <= mycx)
             & (mycx <= lax.shift_right_arithmetic(mix + rr, 4)))
        ids = lax.iota(jnp.int32, 16) + j * 16
        mi = m.astype(jnp.int32)
        pos = plsc.cumsum(mi) - mi + off
        plsc.store_scatter(xl_v, [pos], ids, mask=m)
        return jnp.minimum(off + jnp.sum(mi), P)
    xcnt = lax.fori_loop(0, P // 16, _xscan, jnp.int32(0))

    def _yz(j, _):
        miy = gi_v[pl.ds(P + j * 16, 16)]
        miz = gi_v[pl.ds(2 * P + j * 16, 16)]
        rr = gi_v[pl.ds(3 * P + j * 16, 16)]
        ylo = lax.shift_right_arithmetic(miy - rr, 4)
        yhi = lax.shift_right_arithmetic(miy + rr, 4)
        ym_v[pl.ds(j * 16, 16)] = ((ylo <= cy0) & (cy0 <= yhi)).astype(jnp.int32)
        ym_v[pl.ds(P + j * 16, 16)] = (
            (ylo <= cy0 + 1) & (cy0 + 1 <= yhi)).astype(jnp.int32)
        zlo = lax.shift_right_arithmetic(miz - rr, 4)
        zhi = lax.shift_right_arithmetic(miz + rr, 4)
        for q in range(8):
            zm_v[pl.ds(q * P + j * 16, 16)] = (
                (zlo <= q) & (q <= zhi)).astype(jnp.int32)
        return _
    lax.fori_loop(0, P // 16, _yz, 0)

    nxv = (xcnt + 15) // 16

    def _cell_body(k, _):
        yrow = (k // 8) * P
        zrow = (k - (k // 8) * 8) * P

        def _j(j, off):
            gid = xl_v[pl.ds(j * 16, 16)]
            lane_ok = (lax.iota(jnp.int32, 16) + j * 16) < xcnt
            gid = jnp.where(lane_ok, gid, 0)
            ym = plsc.load_gather(ym_v, [gid + yrow])
            zm = plsc.load_gather(zm_v, [gid + zrow])
            m = lane_ok & ((ym & zm) == 1)
            mi = m.astype(jnp.int32)
            pos = plsc.cumsum(mi) - mi + (k * LSTR + off)
            plsc.store_scatter(ll_v, [pos], gid, mask=m)
            return jnp.minimum(off + jnp.sum(mi), CAP)
        cnt = lax.fori_loop(0, nxv, _j, jnp.int32(0))
        _sstore(lc_v, k, cnt, jnp.int32)
        return _
    lax.fori_loop(0, CPW, _cell_body, 0)

    npts = jnp.int32(0)
    for b in range(N // 2048):
        pltpu.sync_copy(pc_hbm.at[pl.ds(b * 2048, 2048)], cidc_v)

        def _pt_scan(v, np_, _b=b):
            cv = cidc_v[pl.ds(v * 16, 16)]
            m = (cv >> 4) == wid
            ids = lax.iota(jnp.int32, 16) + (_b * 2048 + v * 16)
            mi = m.astype(jnp.int32)
            pos = plsc.cumsum(mi) - mi + np_
            plsc.store_scatter(pidf_v, [pos], ids, mask=m)
            plsc.store_scatter(pcidf_v, [pos], cv, mask=m)
            return jnp.minimum(np_ + jnp.sum(mi), PCAP)
        npts = lax.fori_loop(0, 2048 // 16, _pt_scan, npts)

    def _pt_body(i, hoff):
        pid = _sload(pidf_v, i)
        pcid = _sload(pcidf_v, i)
        k = pcid - wid * CPW
        cnt = _sload(lc_v, k)
        inv_g = 1.0 / GRID
        def _flr(x):
            yi = x.astype(jnp.int32)
            return yi - (yi.astype(jnp.float32) > x).astype(jnp.int32)
        pix = _flr(_sload(pw_v, pid) * inv_g)
        piy = _flr(_sload(pw_v, N + pid) * inv_g)
        piz = _flr(_sload(pw_v, 2 * N + pid) * inv_g)
        stag_v[pl.ds(i * 32, 16)] = jnp.zeros((16,), jnp.float32)
        stag_v[pl.ds(i * 32 + 16, 16)] = jnp.zeros((16,), jnp.float32)
        nj = (cnt + 15) // 16

        def _j(j, hoff):
            idx = ll_v[pl.ds(k * LSTR + j * 16, 16)]
            valid = (lax.iota(jnp.int32, 16) + j * 16) < cnt
            gidx = jnp.where(valid, idx, 0)
            mix = plsc.load_gather(gi_v, [gidx])
            miy = plsc.load_gather(gi_v, [gidx + P])
            miz = plsc.load_gather(gi_v, [gidx + 2 * P])
            rr = plsc.load_gather(gi_v, [gidx + 3 * P])
            mask = (valid
                    & (jnp.abs(pix - mix) <= rr)
                    & (jnp.abs(piy - miy) <= rr)
                    & (jnp.abs(piz - miz) <= rr))
            mi = mask.astype(jnp.int32)
            pos = plsc.cumsum(mi) - mi + hoff
            plsc.store_scatter(hg_v, [pos], gidx, mask=mask)
            plsc.store_scatter(hil_v, [pos],
                               jnp.full((16,), i, jnp.int32), mask=mask)
            return jnp.minimum(hoff + jnp.sum(mi), HITCAP)
        return lax.fori_loop(0, nj, _j, hoff)
    nhits = lax.fori_loop(0, npts, _pt_body, jnp.int32(0))

    nhv = (nhits + 15) // 16

    def _ev(hv, _):
        lane_ok = (lax.iota(jnp.int32, 16) + hv * 16) < nhits
        gidx = jnp.where(lane_ok, hg_v[pl.ds(hv * 16, 16)], 0)
        il = jnp.where(lane_ok, hil_v[pl.ds(hv * 16, 16)], 0)
        pidv = plsc.load_gather(pidf_v, [il])
        px = plsc.load_gather(pw_v, [pidv])
        py = plsc.load_gather(pw_v, [pidv + N])
        pz = plsc.load_gather(pw_v, [pidv + 2 * N])
        mux = plsc.load_gather(gf_v, [gidx])
        muy = plsc.load_gather(gf_v, [gidx + P])
        muz = plsc.load_gather(gf_v, [gidx + 2 * P])
        c0 = plsc.load_gather(gf_v, [gidx + 3 * P])
        c1 = plsc.load_gather(gf_v, [gidx + 4 * P])
        c2 = plsc.load_gather(gf_v, [gidx + 5 * P])
        c3 = plsc.load_gather(gf_v, [gidx + 6 * P])
        c4 = plsc.load_gather(gf_v, [gidx + 7 * P])
        c5 = plsc.load_gather(gf_v, [gidx + 8 * P])
        opg = plsc.load_gather(gf_v, [gidx + 9 * P])
        dx = px - mux
        dy = py - muy
        dz = pz - muz
        power = (-0.5 * (c0 * dx * dx + c1 * dy * dy + c2 * dz * dz)
                 - c3 * dx * dy - c4 * dy * dz - c5 * dx * dz)
        hw_v[pl.ds(hv * 16, 16)] = opg * jnp.exp(power)
        return _
    lax.fori_loop(0, nhv, _ev, 0)

    lo8 = lax.iota(jnp.int32, 16) < 8

    def _hit(h, _):
        wv = _sload(hw_v, h)
        g = _sload(hg_v, h)
        il = _sload(hil_v, h)
        a0 = stag_v[pl.ds(il * 32, 16)]
        stag_v[pl.ds(il * 32, 16)] = a0 + wv * semA_v[pl.ds(g * 16, 16)]
        a1 = stag_v[pl.ds(il * 32 + 16, 16)]
        sb = jnp.where(lo8, semB_v[pl.ds(g * 8, 16)], 0.0)
        stag_v[pl.ds(il * 32 + 16, 16)] = a1 + wv * sb
        return _
    lax.fori_loop(0, nhits, _hit, 0)

    def _out(i, _):
        pid = _sload(pidf_v, i)
        pltpu.async_copy(stag_v.at[pl.ds(i * 32, 32)],
                         out_hbm.at[pl.ds(pid * 32, 32)], dsem)
        return _
    lax.fori_loop(0, npts, _out, 0)

    def _drain(i, _):
        pltpu.make_async_copy(stag_v.at[pl.ds(0, 32)],
                              out_hbm.at[pl.ds(0, 32)], dsem).wait()
        return _
    lax.fori_loop(0, npts, _drain, 0)


def kernel(pts, means3D, opacities, semantics, scales, cov3D, metas, origin_use):
    p = pts[0]
    mu = means3D[0]
    op = opacities[0]
    sem = semantics[0]
    sc = scales[0]
    cov = cov3D[0]

    inv_g = 1.0 / GRID
    pint = jnp.floor((p - origin_use) * inv_g).astype(jnp.int32)
    mint = jnp.floor((mu - origin_use) * inv_g).astype(jnp.int32)
    radii = jnp.ceil(jnp.max(sc, axis=-1) * (SCALE_MULT * inv_g)).astype(jnp.int32)

    pcell = ((pint[:, 0] >> 4) * 64 + (pint[:, 1] >> 4) * 8 + (pint[:, 2] >> 4))

    cov6 = cov.reshape(P, 9)[:, jnp.array([0, 4, 8, 1, 5, 2])]
    gf = jnp.concatenate([(mu - origin_use).T, cov6.T, op[None, :]],
                         axis=0).reshape(-1)
    gi = jnp.concatenate([mint.T, radii[None, :]], axis=0).reshape(-1)
    semA = sem[:, :16].reshape(-1)
    semB = jnp.pad(sem[:, 16:], ((0, 0), (0, 6))).reshape(-1)
    ptf = (p - origin_use).T.reshape(-1)

    out = _sc_agg(gf, gi, semA, semB, ptf, pcell)
    return out.reshape(N, 32)[:, :C]
